# Initial kernel scaffold; baseline (speedup 1.0000x reference)
import jax
import jax.numpy as jnp
from jax.experimental import pallas as pl


def kernel(h, edge_index, W, a_left, a_right):
    raise NotImplementedError("write your pallas kernel here")


# speedup vs baseline: 24.3722x; 24.3722x over previous
import jax
import jax.numpy as jnp
from jax import lax
from jax.experimental import pallas as pl
from jax.experimental.pallas import tpu as pltpu
from jax.experimental.pallas import tpu_sc as plsc

N = 10000
E = 320000
D = 128
DH = D // 2
NC = 2
NS = 16
NW = NC * NS
EW = E // NW
NBLK = 79
EWP = NBLK * 128
NFULL = EW // 128
L = 16
NP = 10240
ROWS_PER_TILE = NP // NS

NEG_BIG = -1.0e30


def _leaky(x):
    return jnp.where(x >= 0.0, x, 0.2 * x)


def _proj_body(h_ref, w_ref, a2_ref, hw_ref, ee_ref):
    h = h_ref[...]
    w = w_ref[...]
    a2 = a2_ref[...]
    hw = lax.dot_general(h, w, (((1,), (1,)), ((), ())),
                         preferred_element_type=jnp.float32)
    hw_ref[:N] = hw[:, :DH]
    hw_ref[N:] = hw[:, DH:]
    a2w = lax.dot_general(a2, w, (((1,), (0,)), ((), ())),
                          preferred_element_type=jnp.float32)
    ee = lax.dot_general(a2w, h, (((1,), (1,)), ((), ())),
                         preferred_element_type=jnp.float32)
    ee_ref[...] = jnp.reshape(ee, (2, 1, N))


def _project(h, W, a2):
    return pl.pallas_call(
        _proj_body,
        out_shape=(
            jax.ShapeDtypeStruct((2 * N, DH), jnp.float32),
            jax.ShapeDtypeStruct((2, 1, N), jnp.float32),
        ),
    )(h, W, a2)


def _logits_body(ee_hbm, src_hbm, dst_hbm, e_hbm, wmax_hbm,
                 el_v, er_v, src_v, dst_v, e_v, wm_v):
    w = lax.axis_index("s") * NC + lax.axis_index("c")
    pltpu.sync_copy(ee_hbm.at[0, 0], el_v)
    pltpu.sync_copy(ee_hbm.at[1, 0], er_v)
    pltpu.sync_copy(src_hbm.at[w], src_v)
    pltpu.sync_copy(dst_hbm.at[w], dst_v)

    def block(b, m):
        for k in range(8):
            sl = pl.ds(k * L, L)
            s = src_v[b, sl]
            d = dst_v[b, sl]
            e = _leaky(plsc.load_gather(el_v, [s]) +
                       plsc.load_gather(er_v, [d]))
            e_v[b, sl] = e
            m = jnp.maximum(m, e)
        return m

    m_v = lax.fori_loop(0, NFULL, block, jnp.full((L,), NEG_BIG, jnp.float32))
    s = src_v[NFULL, pl.ds(0, L)]
    d = dst_v[NFULL, pl.ds(0, L)]
    e = _leaky(plsc.load_gather(el_v, [s]) + plsc.load_gather(er_v, [d]))
    e_v[NFULL, pl.ds(0, L)] = e
    m_v = jnp.maximum(m_v, e)
    for k in range(1, 8):
        e_v[NFULL, pl.ds(k * L, L)] = jnp.full((L,), NEG_BIG, jnp.float32)

    m = jnp.max(m_v)
    pltpu.sync_copy(e_v, e_hbm.at[w])
    wm_v[0, pl.ds(0, L)] = jnp.full((L,), 0.0, jnp.float32) + m
    pltpu.sync_copy(wm_v, wmax_hbm.at[w])


def _logits(ee, srcp, dstp):
    mesh = plsc.VectorSubcoreMesh(core_axis_name="c", subcore_axis_name="s")
    f = pl.kernel(
        _logits_body,
        out_type=(
            jax.ShapeDtypeStruct((NW, NBLK, 128), jnp.float32),
            jax.ShapeDtypeStruct((NW, 1, L), jnp.float32),
        ),
        mesh=mesh,
        scratch_types=[
            pltpu.VMEM((N,), jnp.float32),
            pltpu.VMEM((N,), jnp.float32),
            pltpu.VMEM((NBLK, 128), jnp.int32),
            pltpu.VMEM((NBLK, 128), jnp.int32),
            pltpu.VMEM((NBLK, 128), jnp.float32),
            pltpu.VMEM((1, L), jnp.float32),
        ],
        compiler_params=pltpu.CompilerParams(needs_layout_passes=False),
    )
    return f(ee, srcp, dstp)


def _agg_body(hw_hbm, e_hbm, src_hbm, dst_hbm, wmax_hbm,
              op_hbm, dp_hbm,
              src0_v, src1_v, dst0_v, dst1_v, p0_v, p1_v,
              den_v, rows_v, gidx_v, sidx_v, wm_v,
              sem, out_sp):
    cid = lax.axis_index("c")
    sid = lax.axis_index("s")
    srcs = (src0_v, src1_v)
    dsts = (dst0_v, dst1_v)
    ps = (p0_v, p1_v)

    for c in range(2):
        w = sid * 2 + c
        pltpu.sync_copy(src_hbm.at[w], srcs[c])
        pltpu.sync_copy(dst_hbm.at[w], dsts[c])
        pltpu.sync_copy(e_hbm.at[w], ps[c])
    pltpu.sync_copy(wmax_hbm, wm_v)

    def wmred(i, m):
        return jnp.maximum(m, jnp.max(wm_v[i, 0, pl.ds(0, L)]))
    gmax = lax.fori_loop(0, NW, wmred, jnp.float32(NEG_BIG))

    for c in range(2):
        p_v = ps[c]

        def pexp(i, _):
            b = i // 8
            k = i % 8
            sl = pl.ds(k * L, L)
            p_v[b, sl] = jnp.exp(p_v[b, sl] - gmax)
            return 0
        lax.fori_loop(0, NBLK * 8, pexp, 0)

    def zden(i, _):
        den_v[pl.ds(i * L, L)] = jnp.zeros((L,), jnp.float32)
        return 0
    lax.fori_loop(0, N // L, zden, 0)

    def zrow(i, _):
        r = i // 4
        k = i % 4
        rows_v[r, pl.ds(k * L, L)] = jnp.zeros((L,), jnp.float32)
        return 0
    lax.fori_loop(0, 128 * 4, zrow, 0)
    for j in range(5):
        pltpu.sync_copy(rows_v,
                        out_sp.at[pl.ds(sid * ROWS_PER_TILE + j * 128, 128)])
    plsc.subcore_barrier()

    for c in range(2):
        src_v, dst_v, p_v = srcs[c], dsts[c], ps[c]

        def block(b, _):
            for k in range(8):
                sl = pl.ds(k * L, L)
                gidx_v[sl] = src_v[b, sl] + cid * N
                sidx_v[sl] = dst_v[b, sl]
            pltpu.async_copy(hw_hbm.at[gidx_v], rows_v, sem).wait()

            def scale(g, _):
                pvec = p_v[b, pl.ds(g * L, L)]
                for i in range(L):
                    pi = pvec[i]
                    r = g * L + i
                    for x in range(4):
                        cs = pl.ds(x * L, L)
                        rows_v[r, cs] = rows_v[r, cs] * pi
                return 0
            lax.fori_loop(0, 8, scale, 0)

            @pl.when(cid == 0)
            def _():
                for k in range(8):
                    sl = pl.ds(k * L, L)
                    plsc.addupdate_scatter(den_v, [dst_v[b, sl]],
                                           p_v[b, sl])

            pltpu.sync_copy(rows_v, out_sp.at[sidx_v], add=True)
            return 0

        lax.fori_loop(0, NBLK, block, 0)

    @pl.when(cid == 0)
    def _():
        pltpu.sync_copy(den_v, dp_hbm.at[sid, 0])
    plsc.subcore_barrier()

    for j in range(5):
        r0 = sid * ROWS_PER_TILE + j * 128
        pltpu.sync_copy(out_sp.at[pl.ds(r0, 128)],
                        op_hbm.at[cid, pl.ds(r0, 128)])


def _aggregate(hw, e_buf, srcp, dstp, wmax):
    mesh = plsc.VectorSubcoreMesh(core_axis_name="c", subcore_axis_name="s")
    f = pl.kernel(
        _agg_body,
        out_type=(
            jax.ShapeDtypeStruct((NC, NP, DH), jnp.float32),
            jax.ShapeDtypeStruct((NS, 1, N), jnp.float32),
        ),
        mesh=mesh,
        scratch_types=[
            pltpu.VMEM((NBLK, 128), jnp.int32),
            pltpu.VMEM((NBLK, 128), jnp.int32),
            pltpu.VMEM((NBLK, 128), jnp.int32),
            pltpu.VMEM((NBLK, 128), jnp.int32),
            pltpu.VMEM((NBLK, 128), jnp.float32),
            pltpu.VMEM((NBLK, 128), jnp.float32),
            pltpu.VMEM((N,), jnp.float32),
            pltpu.VMEM((128, DH), jnp.float32),
            pltpu.VMEM((128,), jnp.int32),
            pltpu.VMEM((128,), jnp.int32),
            pltpu.VMEM((NW, 1, L), jnp.float32),
            pltpu.SemaphoreType.DMA,
            pltpu.VMEM_SHARED((NP, DH), jnp.float32),
        ],
        compiler_params=pltpu.CompilerParams(needs_layout_passes=False,
                                             use_tc_tiling_on_sc=False),
    )
    return f(hw, e_buf, srcp, dstp, wmax)


def _norm_body(op_ref, dp_ref, out_ref):
    dp = dp_ref[...][:, 0, :]
    dsum = lax.dot_general(dp, jnp.ones((NS, 1), jnp.float32),
                           (((0,), (0,)), ((), ())),
                           preferred_element_type=jnp.float32)
    dsafe = jnp.where(dsum > 0.0, dsum, 1.0)
    num = jnp.concatenate([op_ref[0, :N, :], op_ref[1, :N, :]], axis=1)
    out_ref[...] = num * (1.0 / dsafe)


def _normalize(op, dp):
    return pl.pallas_call(
        _norm_body,
        out_shape=jax.ShapeDtypeStruct((N, D), jnp.float32),
    )(op, dp)


def kernel(h, edge_index, W, a_left, a_right):
    a2 = jnp.concatenate([a_left, a_right], axis=0)
    src = edge_index[0].reshape(NW, EW)
    dst = edge_index[1].reshape(NW, EW)
    pad = ((0, 0), (0, EWP - EW))
    srcp = jnp.pad(src, pad).reshape(NW, NBLK, 128)
    dstp = jnp.pad(dst, pad).reshape(NW, NBLK, 128)

    hw, ee = _project(h, W, a2)
    e_buf, wmax = _logits(ee, srcp, dstp)
    op, dp = _aggregate(hw, e_buf, srcp, dstp, wmax)
    return _normalize(op, dp)


# speedup vs baseline: 33.1188x; 1.3589x over previous
import jax
import jax.numpy as jnp
from jax import lax
from jax.experimental import pallas as pl
from jax.experimental.pallas import tpu as pltpu
from jax.experimental.pallas import tpu_sc as plsc

N = 10000
E = 320000
D = 128
DH = D // 2
NC = 2
NS = 16
NW = NC * NS
EW = E // NW
NBLK = 79
EWP = NBLK * 128
NFULL = EW // 128
L = 16
NP = 10240
ROWS_PER_TILE = NP // NS

NEG_BIG = -1.0e30


def _leaky(x):
    return jnp.where(x >= 0.0, x, 0.2 * x)


def _proj_body(h_ref, w_ref, a2_ref, hw_ref, ee_ref):
    h = h_ref[...]
    w = w_ref[...]
    a2 = a2_ref[...]
    hw = lax.dot_general(h, w, (((1,), (1,)), ((), ())),
                         preferred_element_type=jnp.float32)
    hw_ref[:N] = hw[:, :DH]
    hw_ref[N:] = hw[:, DH:]
    a2w = lax.dot_general(a2, w, (((1,), (0,)), ((), ())),
                          preferred_element_type=jnp.float32)
    ee = lax.dot_general(a2w, h, (((1,), (1,)), ((), ())),
                         preferred_element_type=jnp.float32)
    ee_ref[...] = jnp.reshape(ee, (2, 1, N))


def _project(h, W, a2):
    return pl.pallas_call(
        _proj_body,
        out_shape=(
            jax.ShapeDtypeStruct((2 * N, DH), jnp.float32),
            jax.ShapeDtypeStruct((2, 1, N), jnp.float32),
        ),
    )(h, W, a2)


def _logits_body(ee_hbm, src_hbm, dst_hbm, e_hbm, wmax_hbm,
                 el_v, er_v, src_v, dst_v, e_v, wm_v):
    w = lax.axis_index("s") * NC + lax.axis_index("c")
    pltpu.sync_copy(ee_hbm.at[0, 0], el_v)
    pltpu.sync_copy(ee_hbm.at[1, 0], er_v)
    pltpu.sync_copy(src_hbm.at[w], src_v)
    pltpu.sync_copy(dst_hbm.at[w], dst_v)

    def block(b, m):
        for k in range(8):
            sl = pl.ds(k * L, L)
            s = src_v[b, sl]
            d = dst_v[b, sl]
            e = _leaky(plsc.load_gather(el_v, [s]) +
                       plsc.load_gather(er_v, [d]))
            e_v[b, sl] = e
            m = jnp.maximum(m, e)
        return m

    m_v = lax.fori_loop(0, NFULL, block, jnp.full((L,), NEG_BIG, jnp.float32))
    s = src_v[NFULL, pl.ds(0, L)]
    d = dst_v[NFULL, pl.ds(0, L)]
    e = _leaky(plsc.load_gather(el_v, [s]) + plsc.load_gather(er_v, [d]))
    e_v[NFULL, pl.ds(0, L)] = e
    m_v = jnp.maximum(m_v, e)
    for k in range(1, 8):
        e_v[NFULL, pl.ds(k * L, L)] = jnp.full((L,), NEG_BIG, jnp.float32)

    m = jnp.max(m_v)
    pltpu.sync_copy(e_v, e_hbm.at[w])
    wm_v[0, pl.ds(0, L)] = jnp.full((L,), 0.0, jnp.float32) + m
    pltpu.sync_copy(wm_v, wmax_hbm.at[w])


def _logits(ee, srcp, dstp):
    mesh = plsc.VectorSubcoreMesh(core_axis_name="c", subcore_axis_name="s")
    f = pl.kernel(
        _logits_body,
        out_type=(
            jax.ShapeDtypeStruct((NW, NBLK, 128), jnp.float32),
            jax.ShapeDtypeStruct((NW, 1, L), jnp.float32),
        ),
        mesh=mesh,
        scratch_types=[
            pltpu.VMEM((N,), jnp.float32),
            pltpu.VMEM((N,), jnp.float32),
            pltpu.VMEM((NBLK, 128), jnp.int32),
            pltpu.VMEM((NBLK, 128), jnp.int32),
            pltpu.VMEM((NBLK, 128), jnp.float32),
            pltpu.VMEM((1, L), jnp.float32),
        ],
        compiler_params=pltpu.CompilerParams(needs_layout_passes=False),
    )
    return f(ee, srcp, dstp)


def _agg_body(hw_hbm, e_hbm, src_hbm, dst_hbm, wmax_hbm,
              op_hbm, dp_hbm,
              src0_v, src1_v, dst0_v, dst1_v, p0_v, p1_v,
              den_v, rows0_v, rows1_v, gidx0_v, gidx1_v, sidx0_v, sidx1_v,
              wm_v, sem0, sem1, out_sp):
    cid = lax.axis_index("c")
    sid = lax.axis_index("s")
    srcs = (src0_v, src1_v)
    dsts = (dst0_v, dst1_v)
    ps = (p0_v, p1_v)

    for c in range(2):
        w = sid * 2 + c
        pltpu.sync_copy(src_hbm.at[w], srcs[c])
        pltpu.sync_copy(dst_hbm.at[w], dsts[c])
        pltpu.sync_copy(e_hbm.at[w], ps[c])
    pltpu.sync_copy(wmax_hbm, wm_v)

    def wmred(i, m):
        return jnp.maximum(m, jnp.max(wm_v[i, 0, pl.ds(0, L)]))
    gmax = lax.fori_loop(0, NW, wmred, jnp.float32(NEG_BIG))

    for c in range(2):
        p_v = ps[c]

        def pexp(i, _):
            b = i // 8
            k = i % 8
            sl = pl.ds(k * L, L)
            p_v[b, sl] = jnp.exp(p_v[b, sl] - gmax)
            return 0
        lax.fori_loop(0, NBLK * 8, pexp, 0)

    def zden(i, _):
        den_v[pl.ds(i * L, L)] = jnp.zeros((L,), jnp.float32)
        return 0
    lax.fori_loop(0, N // L, zden, 0)

    def zrow(i, _):
        r = i // 4
        k = i % 4
        rows0_v[r, pl.ds(k * L, L)] = jnp.zeros((L,), jnp.float32)
        return 0
    lax.fori_loop(0, 128 * 4, zrow, 0)
    for j in range(5):
        pltpu.sync_copy(rows0_v,
                        out_sp.at[pl.ds(sid * ROWS_PER_TILE + j * 128, 128)])
    plsc.subcore_barrier()

    def fire(c, b, g_v, s_v, rows_v, sem):
        src_v, dst_v = srcs[c], dsts[c]
        for k in range(8):
            sl = pl.ds(k * L, L)
            g_v[sl] = src_v[b, sl] + cid * N
            s_v[sl] = dst_v[b, sl]
        pltpu.async_copy(hw_hbm.at[g_v], rows_v, sem)

    def process(c, b, g_v, s_v, rows_v, sem):
        dst_v, p_v = dsts[c], ps[c]
        pltpu.make_async_copy(hw_hbm.at[g_v], rows_v, sem).wait()

        def scale(g, _):
            pvec = p_v[b, pl.ds(g * L, L)]
            for i in range(L):
                pi = pvec[i]
                r = g * L + i
                for x in range(4):
                    cs = pl.ds(x * L, L)
                    rows_v[r, cs] = rows_v[r, cs] * pi
            return 0
        lax.fori_loop(0, 8, scale, 0)

        @pl.when(cid == c)
        def _():
            for k in range(8):
                sl = pl.ds(k * L, L)
                plsc.addupdate_scatter(den_v, [dst_v[b, sl]], p_v[b, sl])

        pltpu.sync_copy(rows_v, out_sp.at[s_v], add=True)

    for c in range(2):
        fire(c, 0, gidx0_v, sidx0_v, rows0_v, sem0)

        def pair(j, _):
            b0 = 2 * j
            fire(c, b0 + 1, gidx1_v, sidx1_v, rows1_v, sem1)
            process(c, b0, gidx0_v, sidx0_v, rows0_v, sem0)
            fire(c, b0 + 2, gidx0_v, sidx0_v, rows0_v, sem0)
            process(c, b0 + 1, gidx1_v, sidx1_v, rows1_v, sem1)
            return 0

        lax.fori_loop(0, (NBLK - 1) // 2, pair, 0)
        process(c, NBLK - 1, gidx0_v, sidx0_v, rows0_v, sem0)

    pltpu.sync_copy(den_v, dp_hbm.at[sid * 2 + cid, 0])
    plsc.subcore_barrier()

    for j in range(5):
        r0 = sid * ROWS_PER_TILE + j * 128
        pltpu.sync_copy(out_sp.at[pl.ds(r0, 128)],
                        op_hbm.at[cid, pl.ds(r0, 128)])


def _aggregate(hw, e_buf, srcp, dstp, wmax):
    mesh = plsc.VectorSubcoreMesh(core_axis_name="c", subcore_axis_name="s")
    f = pl.kernel(
        _agg_body,
        out_type=(
            jax.ShapeDtypeStruct((NC, NP, DH), jnp.float32),
            jax.ShapeDtypeStruct((NW, 1, N), jnp.float32),
        ),
        mesh=mesh,
        scratch_types=[
            pltpu.VMEM((NBLK, 128), jnp.int32),
            pltpu.VMEM((NBLK, 128), jnp.int32),
            pltpu.VMEM((NBLK, 128), jnp.int32),
            pltpu.VMEM((NBLK, 128), jnp.int32),
            pltpu.VMEM((NBLK, 128), jnp.float32),
            pltpu.VMEM((NBLK, 128), jnp.float32),
            pltpu.VMEM((N,), jnp.float32),
            pltpu.VMEM((128, DH), jnp.float32),
            pltpu.VMEM((128, DH), jnp.float32),
            pltpu.VMEM((128,), jnp.int32),
            pltpu.VMEM((128,), jnp.int32),
            pltpu.VMEM((128,), jnp.int32),
            pltpu.VMEM((128,), jnp.int32),
            pltpu.VMEM((NW, 1, L), jnp.float32),
            pltpu.SemaphoreType.DMA,
            pltpu.SemaphoreType.DMA,
            pltpu.VMEM_SHARED((NP, DH), jnp.float32),
        ],
        compiler_params=pltpu.CompilerParams(needs_layout_passes=False,
                                             use_tc_tiling_on_sc=False),
    )
    return f(hw, e_buf, srcp, dstp, wmax)


def _norm_body(op_ref, dp_ref, out_ref):
    dp = dp_ref[...][:, 0, :]
    dsum = lax.dot_general(dp, jnp.ones((NW, 1), jnp.float32),
                           (((0,), (0,)), ((), ())),
                           preferred_element_type=jnp.float32)
    dsafe = jnp.where(dsum > 0.0, dsum, 1.0)
    num = jnp.concatenate([op_ref[0, :N, :], op_ref[1, :N, :]], axis=1)
    out_ref[...] = num * (1.0 / dsafe)


def _normalize(op, dp):
    return pl.pallas_call(
        _norm_body,
        out_shape=jax.ShapeDtypeStruct((N, D), jnp.float32),
    )(op, dp)


def kernel(h, edge_index, W, a_left, a_right):
    a2 = jnp.concatenate([a_left, a_right], axis=0)
    src = edge_index[0].reshape(NW, EW)
    dst = edge_index[1].reshape(NW, EW)
    pad = ((0, 0), (0, EWP - EW))
    srcp = jnp.pad(src, pad).reshape(NW, NBLK, 128)
    dstp = jnp.pad(dst, pad).reshape(NW, NBLK, 128)

    hw, ee = _project(h, W, a2)
    e_buf, wmax = _logits(ee, srcp, dstp)
    op, dp = _aggregate(hw, e_buf, srcp, dstp, wmax)
    return _normalize(op, dp)
